# Initial kernel scaffold; baseline (speedup 1.0000x reference)
import jax
import jax.numpy as jnp
from jax.experimental import pallas as pl


def kernel(x, edge_index1, edge_index2, W1, b1, W2, b2):
    raise NotImplementedError("write your pallas kernel here")


# speedup vs baseline: 4.5479x; 4.5479x over previous
import functools

import jax
import jax.numpy as jnp
from jax import lax
from jax.experimental import pallas as pl
from jax.experimental.pallas import tpu as pltpu
from jax.experimental.pallas import tpu_sc as plsc

NSUB = 16
CHUNK = 80
HCHUNK = 320


def _make_hist(n, e):
    per = e // 8
    nch = per // HCHUNK
    mesh = plsc.VectorSubcoreMesh(core_axis_name="c", subcore_axis_name="s")

    @functools.partial(
        pl.kernel, mesh=mesh,
        compiler_params=pltpu.CompilerParams(needs_layout_passes=False),
        out_type=jax.ShapeDtypeStruct((4 * 8 * n,), jnp.float32),
        scratch_types=[
            pltpu.VMEM((HCHUNK,), jnp.int32),
            pltpu.VMEM((n,), jnp.float32),
        ],
    )
    def k(idx_flat, zeros_n, out, ibuf, hist):
        c = lax.axis_index("c")
        s = lax.axis_index("s")
        hid = 2 * c + s // 8
        sub = s % 8
        pltpu.sync_copy(zeros_n, hist)
        ones = jnp.ones((16,), jnp.float32)
        ibase = hid * e + sub * per

        def body(j, carry):
            pltpu.sync_copy(idx_flat.at[pl.ds(ibase + j * HCHUNK, HCHUNK)],
                            ibuf)
            for v in range(HCHUNK // 16):
                iv = ibuf[pl.ds(v * 16, 16)]
                plsc.addupdate_scatter(hist, [iv], ones)
            return carry

        lax.fori_loop(0, nch, body, 0)
        pltpu.sync_copy(hist, out.at[pl.ds((hid * 8 + sub) * n, n)])

    return k


def _make_agg(n, d, e, edge_split):
    e_per = e // (2 * NSUB if edge_split else NSUB)
    nch = e_per // CHUNK
    r_per = (n // NSUB) // 8 * 8
    r_last = n - (NSUB - 1) * r_per
    mesh = plsc.VectorSubcoreMesh(core_axis_name="c", subcore_axis_name="s")

    def body(tabs, src_h, dst_h, zeros_h,
             out_a, out_b, sidx, didx, rows, acc, sem):
        c = lax.axis_index("c")
        s = lax.axis_index("s")
        rbase = s * r_per
        extra = n - NSUB * r_per
        pltpu.sync_copy(zeros_h.at[pl.ds(rbase, r_per)],
                        acc.at[pl.ds(rbase, r_per)])
        if extra:
            @pl.when(s == NSUB - 1)
            def _():
                pltpu.sync_copy(zeros_h.at[pl.ds(NSUB * r_per, extra)],
                                acc.at[pl.ds(NSUB * r_per, extra)])
        plsc.subcore_barrier()

        base_e = ((c * NSUB + s) if edge_split else s) * e_per

        def phase(tab):
            def chunk(i, carry):
                off = base_e + i * CHUNK
                pltpu.sync_copy(src_h.at[pl.ds(off, CHUNK)], sidx)
                pltpu.sync_copy(dst_h.at[pl.ds(off, CHUNK)], didx)
                pltpu.async_copy(tab.at[sidx], rows, sem).wait()
                pltpu.sync_copy(rows, acc.at[didx], add=True)
                return carry
            lax.fori_loop(0, nch, chunk, 0)

        if edge_split:
            phase(tabs[0])
        else:
            @pl.when(c == 0)
            def _():
                phase(tabs[0])

            @pl.when(c == 1)
            def _():
                phase(tabs[1])

        plsc.subcore_barrier()

        def copy_out(out_ref):
            pltpu.sync_copy(acc.at[pl.ds(rbase, r_per)],
                            out_ref.at[pl.ds(rbase, r_per)])
            if extra:
                @pl.when(s == NSUB - 1)
                def _():
                    pltpu.sync_copy(acc.at[pl.ds(NSUB * r_per, extra)],
                                    out_ref.at[pl.ds(NSUB * r_per, extra)])

        @pl.when(c == 0)
        def _():
            copy_out(out_a)

        @pl.when(c == 1)
        def _():
            copy_out(out_b)

    out_type = (jax.ShapeDtypeStruct((n, d), jnp.float32),
                jax.ShapeDtypeStruct((n, d), jnp.float32))
    scratch = [
        pltpu.VMEM((CHUNK,), jnp.int32),
        pltpu.VMEM((CHUNK,), jnp.int32),
        pltpu.VMEM((CHUNK, d), jnp.float32),
        pltpu.VMEM_SHARED((n, d), jnp.float32),
        pltpu.SemaphoreType.DMA,
    ]
    kw = dict(mesh=mesh, out_type=out_type, scratch_types=scratch)
    if edge_split:
        @functools.partial(pl.kernel, **kw)
        def k(tab, src_h, dst_h, zeros_h, *rest):
            body((tab,), src_h, dst_h, zeros_h, *rest)
    else:
        @functools.partial(pl.kernel, **kw)
        def k(tab_a, tab_b, src_h, dst_h, zeros_h, *rest):
            body((tab_a, tab_b), src_h, dst_h, zeros_h, *rest)
    return k


def _norm_rows(p_ref, rows, r):
    i = pl.program_id(0)
    deg = jnp.sum(p_ref[:, :, pl.ds(i * r, r)], axis=1)
    nrm = lax.rsqrt(jnp.maximum(deg, 1.0))
    return [nrm[j][:, None] for j in rows]


def _l1_body(x_ref, p_ref, w_ref, ha_ref, hb_ref):
    (ns,) = _norm_rows(p_ref, [0], x_ref.shape[0])
    xn = x_ref[...] * ns
    h = jnp.dot(xn, w_ref[...], preferred_element_type=jnp.float32)
    half = h.shape[1] // 2
    ha_ref[...] = h[:, :half]
    hb_ref[...] = h[:, half:]


def _l2_body(aa_ref, ab_ref, p_ref, w_ref, b_ref, o_ref):
    nd, ns = _norm_rows(p_ref, [1, 2], aa_ref.shape[0])
    dhalf = aa_ref.shape[1]
    ta = jnp.maximum(aa_ref[...] * nd + b_ref[0, :dhalf][None, :], 0.0) * ns
    tb = jnp.maximum(ab_ref[...] * nd + b_ref[0, dhalf:][None, :], 0.0) * ns
    w = w_ref[...]
    o_ref[...] = (jnp.dot(ta, w[:dhalf], preferred_element_type=jnp.float32)
                  + jnp.dot(tb, w[dhalf:], preferred_element_type=jnp.float32))


def _l3_body(aa_ref, ab_ref, p_ref, b_ref, o_ref):
    (nd,) = _norm_rows(p_ref, [3], aa_ref.shape[0])
    agg = aa_ref[...] + ab_ref[...]
    o_ref[...] = jnp.maximum(agg * nd + b_ref[0][None, :], 0.0)


def kernel(x, edge_index1, edge_index2, W1, b1, W2, b2):
    n_orig, d_in = x.shape
    e = edge_index1.shape[1]
    d_hid = W1.shape[1]
    d_out = W2.shape[1]
    R = 2048
    n = (n_orig + R - 1) // R * R
    grid = (n // R,)

    x = jnp.pad(x, ((0, n - n_orig), (0, 0)))
    src1 = edge_index1[0].astype(jnp.int32)
    dst1 = edge_index1[1].astype(jnp.int32)
    src2 = edge_index2[0].astype(jnp.int32)
    dst2 = edge_index2[1].astype(jnp.int32)
    idx_all = jnp.concatenate([src1, dst1, src2, dst2])
    zeros_n = jnp.zeros((n,), jnp.float32)
    zeros_nd = jnp.zeros((n, d_hid // 2), jnp.float32)
    b1_2d = b1[None, :]
    b2_2d = b2[None, :]

    parts = _make_hist(n, e)(idx_all, zeros_n).reshape(4, 8, n)

    h1a, h1b = pl.pallas_call(
        _l1_body,
        grid=grid,
        in_specs=[
            pl.BlockSpec((R, d_in), lambda i: (i, 0)),
            pl.BlockSpec((4, 8, n), lambda i: (0, 0, 0)),
            pl.BlockSpec((d_in, d_hid), lambda i: (0, 0)),
        ],
        out_specs=[
            pl.BlockSpec((R, d_hid // 2), lambda i: (i, 0)),
            pl.BlockSpec((R, d_hid // 2), lambda i: (i, 0)),
        ],
        out_shape=[
            jax.ShapeDtypeStruct((n, d_hid // 2), jnp.float32),
            jax.ShapeDtypeStruct((n, d_hid // 2), jnp.float32),
        ],
    )(x, parts, W1)

    a1a, a1b = _make_agg(n, d_hid // 2, e, False)(
        h1a, h1b, src1, dst1, zeros_nd)

    h2 = pl.pallas_call(
        _l2_body,
        grid=grid,
        in_specs=[
            pl.BlockSpec((R, d_hid // 2), lambda i: (i, 0)),
            pl.BlockSpec((R, d_hid // 2), lambda i: (i, 0)),
            pl.BlockSpec((4, 8, n), lambda i: (0, 0, 0)),
            pl.BlockSpec((d_hid, d_out), lambda i: (0, 0)),
            pl.BlockSpec((1, d_hid), lambda i: (0, 0)),
        ],
        out_specs=pl.BlockSpec((R, d_out), lambda i: (i, 0)),
        out_shape=jax.ShapeDtypeStruct((n, d_out), jnp.float32),
    )(a1a, a1b, parts, W2, b1_2d)

    a2a, a2b = _make_agg(n, d_out, e, True)(h2, src2, dst2, zeros_nd)

    out = pl.pallas_call(
        _l3_body,
        grid=grid,
        in_specs=[
            pl.BlockSpec((R, d_out), lambda i: (i, 0)),
            pl.BlockSpec((R, d_out), lambda i: (i, 0)),
            pl.BlockSpec((4, 8, n), lambda i: (0, 0, 0)),
            pl.BlockSpec((1, d_out), lambda i: (0, 0)),
        ],
        out_specs=pl.BlockSpec((R, d_out), lambda i: (i, 0)),
        out_shape=jax.ShapeDtypeStruct((n, d_out), jnp.float32),
    )(a2a, a2b, parts, b2_2d)

    return out[:n_orig]


# speedup vs baseline: 10.9816x; 2.4146x over previous
import functools

import jax
import jax.numpy as jnp
from jax import lax
from jax.experimental import pallas as pl
from jax.experimental.pallas import tpu as pltpu
from jax.experimental.pallas import tpu_sc as plsc

NSUB = 16
CHUNK = 125
HCHUNK = 800


def _make_hist(n, e):
    per = e // 8
    nch = per // HCHUNK
    halfh = nch // 2
    mesh = plsc.VectorSubcoreMesh(core_axis_name="c", subcore_axis_name="s")

    @functools.partial(
        pl.kernel, mesh=mesh,
        compiler_params=pltpu.CompilerParams(needs_layout_passes=False),
        out_type=jax.ShapeDtypeStruct((4 * 8 * n,), jnp.float32),
        scratch_types=[
            pltpu.VMEM((HCHUNK,), jnp.int32),
            pltpu.VMEM((HCHUNK,), jnp.int32),
            pltpu.VMEM((n,), jnp.float32),
            pltpu.SemaphoreType.DMA,
            pltpu.SemaphoreType.DMA,
        ],
    )
    def k(idx_flat, zeros_n, out, ibuf_a, ibuf_b, hist, sem_a, sem_b):
        c = lax.axis_index("c")
        s = lax.axis_index("s")
        hid = 2 * c + s // 8
        sub = s % 8
        pltpu.sync_copy(zeros_n, hist)
        ones = jnp.ones((16,), jnp.float32)
        ibase = hid * e + sub * per
        bufs = (ibuf_a, ibuf_b)
        sems = (sem_a, sem_b)

        def desc(j, b):
            return (idx_flat.at[pl.ds(ibase + j * HCHUNK, HCHUNK)],
                    bufs[b], sems[b])

        def process(buf):
            for v in range(HCHUNK // 16):
                plsc.addupdate_scatter(hist, [buf[pl.ds(v * 16, 16)]], ones)

        pltpu.async_copy(*desc(0, 0))

        def body(j, carry):
            pltpu.async_copy(*desc(2 * j + 1, 1))
            pltpu.make_async_copy(*desc(2 * j, 0)).wait()
            process(ibuf_a)

            @pl.when(j < halfh - 1)
            def _():
                pltpu.async_copy(*desc(2 * j + 2, 0))

            pltpu.make_async_copy(*desc(2 * j + 1, 1)).wait()
            process(ibuf_b)
            return carry

        lax.fori_loop(0, halfh, body, 0)
        pltpu.sync_copy(hist, out.at[pl.ds((hid * 8 + sub) * n, n)])

    return k


def _make_agg(n, d, e, edge_split):
    e_per = e // (2 * NSUB if edge_split else NSUB)
    nch = e_per // CHUNK
    GRP = 16
    ngrp = nch // GRP
    r_per = (n // NSUB) // 8 * 8
    r_last = n - (NSUB - 1) * r_per
    mesh = plsc.VectorSubcoreMesh(core_axis_name="c", subcore_axis_name="s")

    def body(tabs, src_h, dst_h, zeros_h, out_a, out_b,
             sidx, didx, rows_a, rows_b, acc, sem_a, sem_b):
        c = lax.axis_index("c")
        s = lax.axis_index("s")
        rbase = s * r_per
        extra = n - NSUB * r_per
        pltpu.sync_copy(zeros_h.at[pl.ds(rbase, r_per)],
                        acc.at[pl.ds(rbase, r_per)])
        if extra:
            @pl.when(s == NSUB - 1)
            def _():
                pltpu.sync_copy(zeros_h.at[pl.ds(NSUB * r_per, extra)],
                                acc.at[pl.ds(NSUB * r_per, extra)])
        plsc.subcore_barrier()

        row0 = ((c * NSUB + s) if edge_split else s) * nch

        def phase(tab):
            bufs = (rows_a, rows_b)
            sems = (sem_a, sem_b)

            def gather(j, k):
                return pltpu.async_copy(tab.at[sidx.at[j]], bufs[k], sems[k])

            def group(g, carry):
                r = row0 + g * GRP
                pltpu.sync_copy(src_h.at[pl.ds(r, GRP)], sidx)
                pltpu.sync_copy(dst_h.at[pl.ds(r, GRP)], didx)
                h = gather(0, 0)
                for j in range(GRP):
                    nh = gather(j + 1, (j + 1) % 2) if j < GRP - 1 else None
                    h.wait()
                    pltpu.sync_copy(bufs[j % 2], acc.at[didx.at[j]],
                                    add=True)
                    h = nh
                return carry

            lax.fori_loop(0, ngrp, group, 0)

        if edge_split:
            phase(tabs[0])
        else:
            @pl.when(c == 0)
            def _():
                phase(tabs[0])

            @pl.when(c == 1)
            def _():
                phase(tabs[1])

        plsc.subcore_barrier()

        def copy_out(out_ref):
            pltpu.sync_copy(acc.at[pl.ds(rbase, r_per)],
                            out_ref.at[pl.ds(rbase, r_per)])
            if extra:
                @pl.when(s == NSUB - 1)
                def _():
                    pltpu.sync_copy(acc.at[pl.ds(NSUB * r_per, extra)],
                                    out_ref.at[pl.ds(NSUB * r_per, extra)])

        @pl.when(c == 0)
        def _():
            copy_out(out_a)

        @pl.when(c == 1)
        def _():
            copy_out(out_b)

    out_type = (jax.ShapeDtypeStruct((n, d), jnp.float32),
                jax.ShapeDtypeStruct((n, d), jnp.float32))
    scratch = [
        pltpu.VMEM((GRP, CHUNK), jnp.int32),
        pltpu.VMEM((GRP, CHUNK), jnp.int32),
        pltpu.VMEM((CHUNK, d), jnp.float32),
        pltpu.VMEM((CHUNK, d), jnp.float32),
        pltpu.VMEM_SHARED((n, d), jnp.float32),
        pltpu.SemaphoreType.DMA,
        pltpu.SemaphoreType.DMA,
    ]
    kw = dict(mesh=mesh, out_type=out_type, scratch_types=scratch)
    if edge_split:
        @functools.partial(pl.kernel, **kw)
        def k(tab, src_h, dst_h, zeros_h, *rest):
            body((tab,), src_h, dst_h, zeros_h, *rest)
    else:
        @functools.partial(pl.kernel, **kw)
        def k(tab_a, tab_b, src_h, dst_h, zeros_h, *rest):
            body((tab_a, tab_b), src_h, dst_h, zeros_h, *rest)
    return k


def _norm_rows(p_ref, rows, r):
    i = pl.program_id(0)
    deg = jnp.sum(p_ref[:, :, pl.ds(i * r, r)], axis=1)
    nrm = lax.rsqrt(jnp.maximum(deg, 1.0))
    return [nrm[j][:, None] for j in rows]


def _l1_body(x_ref, p_ref, w_ref, ha_ref, hb_ref):
    (ns,) = _norm_rows(p_ref, [0], x_ref.shape[0])
    xn = x_ref[...] * ns
    h = jnp.dot(xn, w_ref[...], preferred_element_type=jnp.float32)
    half = h.shape[1] // 2
    ha_ref[...] = h[:, :half]
    hb_ref[...] = h[:, half:]


def _l2_body(aa_ref, ab_ref, p_ref, w_ref, b_ref, o_ref):
    nd, ns = _norm_rows(p_ref, [1, 2], aa_ref.shape[0])
    dhalf = aa_ref.shape[1]
    ta = jnp.maximum(aa_ref[...] * nd + b_ref[0, :dhalf][None, :], 0.0) * ns
    tb = jnp.maximum(ab_ref[...] * nd + b_ref[0, dhalf:][None, :], 0.0) * ns
    w = w_ref[...]
    o_ref[...] = (jnp.dot(ta, w[:dhalf], preferred_element_type=jnp.float32)
                  + jnp.dot(tb, w[dhalf:], preferred_element_type=jnp.float32))


def _l3_body(aa_ref, ab_ref, p_ref, b_ref, o_ref):
    (nd,) = _norm_rows(p_ref, [3], aa_ref.shape[0])
    agg = aa_ref[...] + ab_ref[...]
    o_ref[...] = jnp.maximum(agg * nd + b_ref[0][None, :], 0.0)


def kernel(x, edge_index1, edge_index2, W1, b1, W2, b2):
    n_orig, d_in = x.shape
    e = edge_index1.shape[1]
    d_hid = W1.shape[1]
    d_out = W2.shape[1]
    R = 2048
    n = (n_orig + R - 1) // R * R
    grid = (n // R,)

    x = jnp.pad(x, ((0, n - n_orig), (0, 0)))
    src1 = edge_index1[0].astype(jnp.int32)
    dst1 = edge_index1[1].astype(jnp.int32)
    src2 = edge_index2[0].astype(jnp.int32)
    dst2 = edge_index2[1].astype(jnp.int32)
    idx_all = jnp.concatenate([src1, dst1, src2, dst2])
    zeros_n = jnp.zeros((n,), jnp.float32)
    zeros_nd = jnp.zeros((n, d_hid // 2), jnp.float32)
    b1_2d = b1[None, :]
    b2_2d = b2[None, :]

    parts = _make_hist(n, e)(idx_all, zeros_n).reshape(4, 8, n)

    h1a, h1b = pl.pallas_call(
        _l1_body,
        grid=grid,
        in_specs=[
            pl.BlockSpec((R, d_in), lambda i: (i, 0)),
            pl.BlockSpec((4, 8, n), lambda i: (0, 0, 0)),
            pl.BlockSpec((d_in, d_hid), lambda i: (0, 0)),
        ],
        out_specs=[
            pl.BlockSpec((R, d_hid // 2), lambda i: (i, 0)),
            pl.BlockSpec((R, d_hid // 2), lambda i: (i, 0)),
        ],
        out_shape=[
            jax.ShapeDtypeStruct((n, d_hid // 2), jnp.float32),
            jax.ShapeDtypeStruct((n, d_hid // 2), jnp.float32),
        ],
    )(x, parts, W1)

    a1a, a1b = _make_agg(n, d_hid // 2, e, False)(
        h1a, h1b, src1.reshape(-1, CHUNK), dst1.reshape(-1, CHUNK),
        zeros_nd)

    h2 = pl.pallas_call(
        _l2_body,
        grid=grid,
        in_specs=[
            pl.BlockSpec((R, d_hid // 2), lambda i: (i, 0)),
            pl.BlockSpec((R, d_hid // 2), lambda i: (i, 0)),
            pl.BlockSpec((4, 8, n), lambda i: (0, 0, 0)),
            pl.BlockSpec((d_hid, d_out), lambda i: (0, 0)),
            pl.BlockSpec((1, d_hid), lambda i: (0, 0)),
        ],
        out_specs=pl.BlockSpec((R, d_out), lambda i: (i, 0)),
        out_shape=jax.ShapeDtypeStruct((n, d_out), jnp.float32),
    )(a1a, a1b, parts, W2, b1_2d)

    a2a, a2b = _make_agg(n, d_out, e, True)(
        h2, src2.reshape(-1, CHUNK), dst2.reshape(-1, CHUNK), zeros_nd)

    out = pl.pallas_call(
        _l3_body,
        grid=grid,
        in_specs=[
            pl.BlockSpec((R, d_out), lambda i: (i, 0)),
            pl.BlockSpec((R, d_out), lambda i: (i, 0)),
            pl.BlockSpec((4, 8, n), lambda i: (0, 0, 0)),
            pl.BlockSpec((1, d_out), lambda i: (0, 0)),
        ],
        out_specs=pl.BlockSpec((R, d_out), lambda i: (i, 0)),
        out_shape=jax.ShapeDtypeStruct((n, d_out), jnp.float32),
    )(a2a, a2b, parts, b2_2d)

    return out[:n_orig]


# speedup vs baseline: 10.9937x; 1.0011x over previous
import functools

import jax
import jax.numpy as jnp
from jax import lax
from jax.experimental import pallas as pl
from jax.experimental.pallas import tpu as pltpu
from jax.experimental.pallas import tpu_sc as plsc

NSUB = 16
CHUNK = 125
HCHUNK = 800


def _make_hist(n, e):
    per = e // 8
    nch = per // HCHUNK
    halfh = nch // 2
    mesh = plsc.VectorSubcoreMesh(core_axis_name="c", subcore_axis_name="s")

    @functools.partial(
        pl.kernel, mesh=mesh,
        compiler_params=pltpu.CompilerParams(needs_layout_passes=False),
        out_type=jax.ShapeDtypeStruct((4 * 8 * n,), jnp.float32),
        scratch_types=[
            pltpu.VMEM((HCHUNK,), jnp.int32),
            pltpu.VMEM((HCHUNK,), jnp.int32),
            pltpu.VMEM((n,), jnp.float32),
            pltpu.SemaphoreType.DMA,
            pltpu.SemaphoreType.DMA,
        ],
    )
    def k(idx_flat, zeros_n, out, ibuf_a, ibuf_b, hist, sem_a, sem_b):
        c = lax.axis_index("c")
        s = lax.axis_index("s")
        hid = 2 * c + s // 8
        sub = s % 8
        pltpu.sync_copy(zeros_n, hist)
        ones = jnp.ones((16,), jnp.float32)
        ibase = hid * e + sub * per
        bufs = (ibuf_a, ibuf_b)
        sems = (sem_a, sem_b)

        def desc(j, b):
            return (idx_flat.at[pl.ds(ibase + j * HCHUNK, HCHUNK)],
                    bufs[b], sems[b])

        def process(buf):
            for v in range(HCHUNK // 16):
                plsc.addupdate_scatter(hist, [buf[pl.ds(v * 16, 16)]], ones)

        pltpu.async_copy(*desc(0, 0))

        def body(j, carry):
            pltpu.async_copy(*desc(2 * j + 1, 1))
            pltpu.make_async_copy(*desc(2 * j, 0)).wait()
            process(ibuf_a)

            @pl.when(j < halfh - 1)
            def _():
                pltpu.async_copy(*desc(2 * j + 2, 0))

            pltpu.make_async_copy(*desc(2 * j + 1, 1)).wait()
            process(ibuf_b)
            return carry

        lax.fori_loop(0, halfh, body, 0)
        pltpu.sync_copy(hist, out.at[pl.ds((hid * 8 + sub) * n, n)])

    return k


def _make_agg(n, d, e, edge_split):
    e_per = e // (2 * NSUB if edge_split else NSUB)
    nch = e_per // CHUNK
    GRP = 16
    ngrp = nch // GRP
    r_per = (n // NSUB) // 8 * 8
    r_last = n - (NSUB - 1) * r_per
    mesh = plsc.VectorSubcoreMesh(core_axis_name="c", subcore_axis_name="s")

    def body(tabs, src_h, dst_h, zeros_h, out_a, out_b,
             sidx, didx, rows_a, rows_b, acc,
             sem_a, sem_b, ssem_a, ssem_b):
        c = lax.axis_index("c")
        s = lax.axis_index("s")
        rbase = s * r_per
        extra = n - NSUB * r_per
        pltpu.sync_copy(zeros_h.at[pl.ds(rbase, r_per)],
                        acc.at[pl.ds(rbase, r_per)])
        if extra:
            @pl.when(s == NSUB - 1)
            def _():
                pltpu.sync_copy(zeros_h.at[pl.ds(NSUB * r_per, extra)],
                                acc.at[pl.ds(NSUB * r_per, extra)])
        plsc.subcore_barrier()

        row0 = ((c * NSUB + s) if edge_split else s) * nch

        def phase(tab):
            bufs = (rows_a, rows_b)
            sems = (sem_a, sem_b)
            ssems = (ssem_a, ssem_b)

            def gather(j, k):
                return pltpu.async_copy(tab.at[sidx.at[j]], bufs[k], sems[k])

            def group(g, carry):
                r = row0 + g * GRP
                pltpu.sync_copy(src_h.at[pl.ds(r, GRP)], sidx)
                pltpu.sync_copy(dst_h.at[pl.ds(r, GRP)], didx)
                h = gather(0, 0)
                sc = [None, None]
                for j in range(GRP):
                    k = j % 2
                    nk = (j + 1) % 2
                    if j < GRP - 1:
                        if sc[nk] is not None:
                            sc[nk].wait()
                        nh = gather(j + 1, nk)
                    else:
                        nh = None
                    h.wait()
                    sc[k] = pltpu.async_copy(bufs[k], acc.at[didx.at[j]],
                                             ssems[k], add=True)
                    h = nh
                sc[(GRP - 2) % 2].wait()
                sc[(GRP - 1) % 2].wait()
                return carry

            lax.fori_loop(0, ngrp, group, 0)

        if edge_split:
            phase(tabs[0])
        else:
            @pl.when(c == 0)
            def _():
                phase(tabs[0])

            @pl.when(c == 1)
            def _():
                phase(tabs[1])

        plsc.subcore_barrier()

        def copy_out(out_ref):
            pltpu.sync_copy(acc.at[pl.ds(rbase, r_per)],
                            out_ref.at[pl.ds(rbase, r_per)])
            if extra:
                @pl.when(s == NSUB - 1)
                def _():
                    pltpu.sync_copy(acc.at[pl.ds(NSUB * r_per, extra)],
                                    out_ref.at[pl.ds(NSUB * r_per, extra)])

        @pl.when(c == 0)
        def _():
            copy_out(out_a)

        @pl.when(c == 1)
        def _():
            copy_out(out_b)

    out_type = (jax.ShapeDtypeStruct((n, d), jnp.float32),
                jax.ShapeDtypeStruct((n, d), jnp.float32))
    scratch = [
        pltpu.VMEM((GRP, CHUNK), jnp.int32),
        pltpu.VMEM((GRP, CHUNK), jnp.int32),
        pltpu.VMEM((CHUNK, d), jnp.float32),
        pltpu.VMEM((CHUNK, d), jnp.float32),
        pltpu.VMEM_SHARED((n, d), jnp.float32),
        pltpu.SemaphoreType.DMA,
        pltpu.SemaphoreType.DMA,
        pltpu.SemaphoreType.DMA,
        pltpu.SemaphoreType.DMA,
    ]
    kw = dict(mesh=mesh, out_type=out_type, scratch_types=scratch)
    if edge_split:
        @functools.partial(pl.kernel, **kw)
        def k(tab, src_h, dst_h, zeros_h, *rest):
            body((tab,), src_h, dst_h, zeros_h, *rest)
    else:
        @functools.partial(pl.kernel, **kw)
        def k(tab_a, tab_b, src_h, dst_h, zeros_h, *rest):
            body((tab_a, tab_b), src_h, dst_h, zeros_h, *rest)
    return k


def _norm_rows(p_ref, rows, r):
    i = pl.program_id(0)
    deg = jnp.sum(p_ref[:, :, pl.ds(i * r, r)], axis=1)
    nrm = lax.rsqrt(jnp.maximum(deg, 1.0))
    return [nrm[j][:, None] for j in rows]


def _l1_body(x_ref, p_ref, w_ref, ha_ref, hb_ref):
    (ns,) = _norm_rows(p_ref, [0], x_ref.shape[0])
    xn = x_ref[...] * ns
    h = jnp.dot(xn, w_ref[...], preferred_element_type=jnp.float32)
    half = h.shape[1] // 2
    ha_ref[...] = h[:, :half]
    hb_ref[...] = h[:, half:]


def _l2_body(aa_ref, ab_ref, p_ref, w_ref, b_ref, o_ref):
    nd, ns = _norm_rows(p_ref, [1, 2], aa_ref.shape[0])
    dhalf = aa_ref.shape[1]
    ta = jnp.maximum(aa_ref[...] * nd + b_ref[0, :dhalf][None, :], 0.0) * ns
    tb = jnp.maximum(ab_ref[...] * nd + b_ref[0, dhalf:][None, :], 0.0) * ns
    w = w_ref[...]
    o_ref[...] = (jnp.dot(ta, w[:dhalf], preferred_element_type=jnp.float32)
                  + jnp.dot(tb, w[dhalf:], preferred_element_type=jnp.float32))


def _l3_body(aa_ref, ab_ref, p_ref, b_ref, o_ref):
    (nd,) = _norm_rows(p_ref, [3], aa_ref.shape[0])
    agg = aa_ref[...] + ab_ref[...]
    o_ref[...] = jnp.maximum(agg * nd + b_ref[0][None, :], 0.0)


def kernel(x, edge_index1, edge_index2, W1, b1, W2, b2):
    n_orig, d_in = x.shape
    e = edge_index1.shape[1]
    d_hid = W1.shape[1]
    d_out = W2.shape[1]
    R = 2048
    n = (n_orig + R - 1) // R * R
    grid = (n // R,)

    x = jnp.pad(x, ((0, n - n_orig), (0, 0)))
    src1 = edge_index1[0].astype(jnp.int32)
    dst1 = edge_index1[1].astype(jnp.int32)
    src2 = edge_index2[0].astype(jnp.int32)
    dst2 = edge_index2[1].astype(jnp.int32)
    idx_all = jnp.concatenate([src1, dst1, src2, dst2])
    zeros_n = jnp.zeros((n,), jnp.float32)
    zeros_nd = jnp.zeros((n, d_hid // 2), jnp.float32)
    b1_2d = b1[None, :]
    b2_2d = b2[None, :]

    parts = _make_hist(n, e)(idx_all, zeros_n).reshape(4, 8, n)

    h1a, h1b = pl.pallas_call(
        _l1_body,
        grid=grid,
        in_specs=[
            pl.BlockSpec((R, d_in), lambda i: (i, 0)),
            pl.BlockSpec((4, 8, n), lambda i: (0, 0, 0)),
            pl.BlockSpec((d_in, d_hid), lambda i: (0, 0)),
        ],
        out_specs=[
            pl.BlockSpec((R, d_hid // 2), lambda i: (i, 0)),
            pl.BlockSpec((R, d_hid // 2), lambda i: (i, 0)),
        ],
        out_shape=[
            jax.ShapeDtypeStruct((n, d_hid // 2), jnp.float32),
            jax.ShapeDtypeStruct((n, d_hid // 2), jnp.float32),
        ],
    )(x, parts, W1)

    a1a, a1b = _make_agg(n, d_hid // 2, e, False)(
        h1a, h1b, src1.reshape(-1, CHUNK), dst1.reshape(-1, CHUNK),
        zeros_nd)

    h2 = pl.pallas_call(
        _l2_body,
        grid=grid,
        in_specs=[
            pl.BlockSpec((R, d_hid // 2), lambda i: (i, 0)),
            pl.BlockSpec((R, d_hid // 2), lambda i: (i, 0)),
            pl.BlockSpec((4, 8, n), lambda i: (0, 0, 0)),
            pl.BlockSpec((d_hid, d_out), lambda i: (0, 0)),
            pl.BlockSpec((1, d_hid), lambda i: (0, 0)),
        ],
        out_specs=pl.BlockSpec((R, d_out), lambda i: (i, 0)),
        out_shape=jax.ShapeDtypeStruct((n, d_out), jnp.float32),
    )(a1a, a1b, parts, W2, b1_2d)

    a2a, a2b = _make_agg(n, d_out, e, True)(
        h2, src2.reshape(-1, CHUNK), dst2.reshape(-1, CHUNK), zeros_nd)

    out = pl.pallas_call(
        _l3_body,
        grid=grid,
        in_specs=[
            pl.BlockSpec((R, d_out), lambda i: (i, 0)),
            pl.BlockSpec((R, d_out), lambda i: (i, 0)),
            pl.BlockSpec((4, 8, n), lambda i: (0, 0, 0)),
            pl.BlockSpec((1, d_out), lambda i: (0, 0)),
        ],
        out_specs=pl.BlockSpec((R, d_out), lambda i: (i, 0)),
        out_shape=jax.ShapeDtypeStruct((n, d_out), jnp.float32),
    )(a2a, a2b, parts, b2_2d)

    return out[:n_orig]


# speedup vs baseline: 11.7512x; 1.0689x over previous
import functools

import jax
import jax.numpy as jnp
from jax import lax
from jax.experimental import pallas as pl
from jax.experimental.pallas import tpu as pltpu
from jax.experimental.pallas import tpu_sc as plsc

NSUB = 16
CHUNK = 125
HCHUNK = 800


def _make_hist(n, e):
    per = e // 8
    nch = per // HCHUNK
    halfh = nch // 2
    mesh = plsc.VectorSubcoreMesh(core_axis_name="c", subcore_axis_name="s")

    @functools.partial(
        pl.kernel, mesh=mesh,
        compiler_params=pltpu.CompilerParams(needs_layout_passes=False),
        out_type=jax.ShapeDtypeStruct((4 * 8 * n,), jnp.float32),
        scratch_types=[
            pltpu.VMEM((HCHUNK,), jnp.int32),
            pltpu.VMEM((HCHUNK,), jnp.int32),
            pltpu.VMEM((n,), jnp.float32),
            pltpu.SemaphoreType.DMA,
            pltpu.SemaphoreType.DMA,
        ],
    )
    def k(idx_flat, zeros_n, out, ibuf_a, ibuf_b, hist, sem_a, sem_b):
        c = lax.axis_index("c")
        s = lax.axis_index("s")
        hid = 2 * c + s // 8
        sub = s % 8
        pltpu.sync_copy(zeros_n, hist)
        ones = jnp.ones((16,), jnp.float32)
        ibase = hid * e + sub * per
        bufs = (ibuf_a, ibuf_b)
        sems = (sem_a, sem_b)

        def desc(j, b):
            return (idx_flat.at[pl.ds(ibase + j * HCHUNK, HCHUNK)],
                    bufs[b], sems[b])

        def process(buf):
            for v in range(HCHUNK // 16):
                plsc.addupdate_scatter(hist, [buf[pl.ds(v * 16, 16)]], ones)

        pltpu.async_copy(*desc(0, 0))

        def body(j, carry):
            pltpu.async_copy(*desc(2 * j + 1, 1))
            pltpu.make_async_copy(*desc(2 * j, 0)).wait()
            process(ibuf_a)

            @pl.when(j < halfh - 1)
            def _():
                pltpu.async_copy(*desc(2 * j + 2, 0))

            pltpu.make_async_copy(*desc(2 * j + 1, 1)).wait()
            process(ibuf_b)
            return carry

        lax.fori_loop(0, halfh, body, 0)
        pltpu.sync_copy(hist, out.at[pl.ds((hid * 8 + sub) * n, n)])

    return k


def _make_agg(n, d, e, edge_split):
    e_per = e // (2 * NSUB if edge_split else NSUB)
    nch = e_per // CHUNK
    GRP = 16
    ngrp = nch // GRP
    r_per = (n // NSUB) // 8 * 8
    r_last = n - (NSUB - 1) * r_per
    mesh = plsc.VectorSubcoreMesh(core_axis_name="c", subcore_axis_name="s")

    def body(tabs, src_h, dst_h, zeros_h, out_a, out_b,
             sidx, didx, rows_a, rows_b, acc,
             sem_a, sem_b, ssem_a, ssem_b, isem_s, isem_d):
        c = lax.axis_index("c")
        s = lax.axis_index("s")
        rbase = s * r_per
        extra = n - NSUB * r_per
        pltpu.sync_copy(zeros_h.at[pl.ds(rbase, r_per)],
                        acc.at[pl.ds(rbase, r_per)])
        if extra:
            @pl.when(s == NSUB - 1)
            def _():
                pltpu.sync_copy(zeros_h.at[pl.ds(NSUB * r_per, extra)],
                                acc.at[pl.ds(NSUB * r_per, extra)])
        plsc.subcore_barrier()

        row0 = ((c * NSUB + s) if edge_split else s) * nch

        def phase(tab):
            bufs = (rows_a, rows_b)
            sems = (sem_a, sem_b)
            ssems = (ssem_a, ssem_b)

            def idx_start(g1):
                r = row0 + g1 * GRP
                pltpu.async_copy(src_h.at[pl.ds(r, GRP)], sidx.at[g1 % 2],
                                 isem_s)
                pltpu.async_copy(dst_h.at[pl.ds(r, GRP)], didx.at[g1 % 3],
                                 isem_d)

            def idx_wait(g1):
                r = row0 + g1 * GRP
                pltpu.make_async_copy(src_h.at[pl.ds(r, GRP)],
                                      sidx.at[g1 % 2], isem_s).wait()
                pltpu.make_async_copy(dst_h.at[pl.ds(r, GRP)],
                                      didx.at[g1 % 3], isem_d).wait()

            idx_start(0)

            def group(g, carry):
                slot = g % 2
                dslot = g % 3
                idx_wait(g)

                @pl.when(g < ngrp - 1)
                def _():
                    idx_start(g + 1)

                def gather(j, k):
                    return pltpu.async_copy(tab.at[sidx.at[slot, j]],
                                            bufs[k], sems[k])

                def prev_wait(k):
                    @pl.when(g > 0)
                    def _():
                        pltpu.make_async_copy(
                            bufs[k],
                            acc.at[didx.at[(g + 2) % 3, GRP - 2 + k]],
                            ssems[k]).wait()

                prev_wait(0)
                h = gather(0, 0)
                sc = [None, None]
                for j in range(GRP):
                    k = j % 2
                    nk = (j + 1) % 2
                    if j < GRP - 1:
                        if sc[nk] is not None:
                            sc[nk].wait()
                        else:
                            prev_wait(nk)
                        nh = gather(j + 1, nk)
                    else:
                        nh = None
                    h.wait()
                    sc[k] = pltpu.async_copy(bufs[k],
                                             acc.at[didx.at[dslot, j]],
                                             ssems[k], add=True)
                    h = nh
                return carry

            lax.fori_loop(0, ngrp, group, 0)
            lslot = (ngrp - 1) % 3
            pltpu.make_async_copy(bufs[0], acc.at[didx.at[lslot, GRP - 2]],
                                  ssems[0]).wait()
            pltpu.make_async_copy(bufs[1], acc.at[didx.at[lslot, GRP - 1]],
                                  ssems[1]).wait()

        if edge_split:
            phase(tabs[0])
        else:
            @pl.when(c == 0)
            def _():
                phase(tabs[0])

            @pl.when(c == 1)
            def _():
                phase(tabs[1])

        plsc.subcore_barrier()

        def copy_out(out_ref):
            pltpu.sync_copy(acc.at[pl.ds(rbase, r_per)],
                            out_ref.at[pl.ds(rbase, r_per)])
            if extra:
                @pl.when(s == NSUB - 1)
                def _():
                    pltpu.sync_copy(acc.at[pl.ds(NSUB * r_per, extra)],
                                    out_ref.at[pl.ds(NSUB * r_per, extra)])

        @pl.when(c == 0)
        def _():
            copy_out(out_a)

        @pl.when(c == 1)
        def _():
            copy_out(out_b)

    out_type = (jax.ShapeDtypeStruct((n, d), jnp.float32),
                jax.ShapeDtypeStruct((n, d), jnp.float32))
    scratch = [
        pltpu.VMEM((2, GRP, CHUNK), jnp.int32),
        pltpu.VMEM((3, GRP, CHUNK), jnp.int32),
        pltpu.VMEM((CHUNK, d), jnp.float32),
        pltpu.VMEM((CHUNK, d), jnp.float32),
        pltpu.VMEM_SHARED((n, d), jnp.float32),
        pltpu.SemaphoreType.DMA,
        pltpu.SemaphoreType.DMA,
        pltpu.SemaphoreType.DMA,
        pltpu.SemaphoreType.DMA,
        pltpu.SemaphoreType.DMA,
        pltpu.SemaphoreType.DMA,
    ]
    kw = dict(mesh=mesh, out_type=out_type, scratch_types=scratch)
    if edge_split:
        @functools.partial(pl.kernel, **kw)
        def k(tab, src_h, dst_h, zeros_h, *rest):
            body((tab,), src_h, dst_h, zeros_h, *rest)
    else:
        @functools.partial(pl.kernel, **kw)
        def k(tab_a, tab_b, src_h, dst_h, zeros_h, *rest):
            body((tab_a, tab_b), src_h, dst_h, zeros_h, *rest)
    return k


def _norm_rows(p_ref, rows, r):
    i = pl.program_id(0)
    deg = jnp.sum(p_ref[:, :, pl.ds(i * r, r)], axis=1)
    nrm = lax.rsqrt(jnp.maximum(deg, 1.0))
    return [nrm[j][:, None] for j in rows]


def _l1_body(x_ref, p_ref, w_ref, ha_ref, hb_ref):
    (ns,) = _norm_rows(p_ref, [0], x_ref.shape[0])
    xn = x_ref[...] * ns
    h = jnp.dot(xn, w_ref[...], preferred_element_type=jnp.float32)
    half = h.shape[1] // 2
    ha_ref[...] = h[:, :half]
    hb_ref[...] = h[:, half:]


def _l2_body(aa_ref, ab_ref, p_ref, w_ref, b_ref, o_ref):
    nd, ns = _norm_rows(p_ref, [1, 2], aa_ref.shape[0])
    dhalf = aa_ref.shape[1]
    ta = jnp.maximum(aa_ref[...] * nd + b_ref[0, :dhalf][None, :], 0.0) * ns
    tb = jnp.maximum(ab_ref[...] * nd + b_ref[0, dhalf:][None, :], 0.0) * ns
    w = w_ref[...]
    o_ref[...] = (jnp.dot(ta, w[:dhalf], preferred_element_type=jnp.float32)
                  + jnp.dot(tb, w[dhalf:], preferred_element_type=jnp.float32))


def _l3_body(aa_ref, ab_ref, p_ref, b_ref, o_ref):
    (nd,) = _norm_rows(p_ref, [3], aa_ref.shape[0])
    agg = aa_ref[...] + ab_ref[...]
    o_ref[...] = jnp.maximum(agg * nd + b_ref[0][None, :], 0.0)


def kernel(x, edge_index1, edge_index2, W1, b1, W2, b2):
    n_orig, d_in = x.shape
    e = edge_index1.shape[1]
    d_hid = W1.shape[1]
    d_out = W2.shape[1]
    R = 2048
    n = (n_orig + R - 1) // R * R
    grid = (n // R,)

    x = jnp.pad(x, ((0, n - n_orig), (0, 0)))
    src1 = edge_index1[0].astype(jnp.int32)
    dst1 = edge_index1[1].astype(jnp.int32)
    src2 = edge_index2[0].astype(jnp.int32)
    dst2 = edge_index2[1].astype(jnp.int32)
    idx_all = jnp.concatenate([src1, dst1, src2, dst2])
    zeros_n = jnp.zeros((n,), jnp.float32)
    zeros_nd = jnp.zeros((n, d_hid // 2), jnp.float32)
    b1_2d = b1[None, :]
    b2_2d = b2[None, :]

    parts = _make_hist(n, e)(idx_all, zeros_n).reshape(4, 8, n)

    h1a, h1b = pl.pallas_call(
        _l1_body,
        grid=grid,
        in_specs=[
            pl.BlockSpec((R, d_in), lambda i: (i, 0)),
            pl.BlockSpec((4, 8, n), lambda i: (0, 0, 0)),
            pl.BlockSpec((d_in, d_hid), lambda i: (0, 0)),
        ],
        out_specs=[
            pl.BlockSpec((R, d_hid // 2), lambda i: (i, 0)),
            pl.BlockSpec((R, d_hid // 2), lambda i: (i, 0)),
        ],
        out_shape=[
            jax.ShapeDtypeStruct((n, d_hid // 2), jnp.float32),
            jax.ShapeDtypeStruct((n, d_hid // 2), jnp.float32),
        ],
    )(x, parts, W1)

    a1a, a1b = _make_agg(n, d_hid // 2, e, False)(
        h1a, h1b, src1.reshape(-1, CHUNK), dst1.reshape(-1, CHUNK),
        zeros_nd)

    h2 = pl.pallas_call(
        _l2_body,
        grid=grid,
        in_specs=[
            pl.BlockSpec((R, d_hid // 2), lambda i: (i, 0)),
            pl.BlockSpec((R, d_hid // 2), lambda i: (i, 0)),
            pl.BlockSpec((4, 8, n), lambda i: (0, 0, 0)),
            pl.BlockSpec((d_hid, d_out), lambda i: (0, 0)),
            pl.BlockSpec((1, d_hid), lambda i: (0, 0)),
        ],
        out_specs=pl.BlockSpec((R, d_out), lambda i: (i, 0)),
        out_shape=jax.ShapeDtypeStruct((n, d_out), jnp.float32),
    )(a1a, a1b, parts, W2, b1_2d)

    a2a, a2b = _make_agg(n, d_out, e, True)(
        h2, src2.reshape(-1, CHUNK), dst2.reshape(-1, CHUNK), zeros_nd)

    out = pl.pallas_call(
        _l3_body,
        grid=grid,
        in_specs=[
            pl.BlockSpec((R, d_out), lambda i: (i, 0)),
            pl.BlockSpec((R, d_out), lambda i: (i, 0)),
            pl.BlockSpec((4, 8, n), lambda i: (0, 0, 0)),
            pl.BlockSpec((1, d_out), lambda i: (0, 0)),
        ],
        out_specs=pl.BlockSpec((R, d_out), lambda i: (i, 0)),
        out_shape=jax.ShapeDtypeStruct((n, d_out), jnp.float32),
    )(a2a, a2b, parts, b2_2d)

    return out[:n_orig]
